# initial kernel scaffold (unmeasured)
import jax
import jax.numpy as jnp
from jax import lax
from jax.experimental import pallas as pl
from jax.experimental.pallas import tpu as pltpu

N_DEV = 32


def kernel(A, B):
    m_per, k = A.shape
    _, n = B.shape
    M = N_DEV * m_per

    def body(a_ref, b_ref, out_ref, gat_ref, cbuf_ref, send_sems, recv_sems, out_sems):
        my_pos = lax.axis_index("i")
        left = lax.rem(my_pos + N_DEV - 1, N_DEV)
        right = lax.rem(my_pos + 1, N_DEV)

        barrier_sem = pltpu.get_barrier_semaphore()
        for nbr in (left, right):
            pl.semaphore_signal(
                barrier_sem, inc=1,
                device_id=(nbr,), device_id_type=pl.DeviceIdType.MESH,
            )
        pl.semaphore_wait(barrier_sem, 2)

        gat_ref[0, :, :] = a_ref[:, :]

        def compute_and_store(slot, cslot):
            origin = lax.rem(my_pos + N_DEV - slot, N_DEV)
            cbuf_ref[cslot, :, :] = jnp.dot(
                gat_ref[slot, :, :], b_ref[:, :],
                preferred_element_type=jnp.float32,
            )
            cp = pltpu.make_async_copy(
                cbuf_ref.at[cslot],
                out_ref.at[pl.ds(origin * m_per, m_per), :],
                out_sems.at[cslot],
            )
            cp.start()
            cp.wait()

        compute_and_store(0, 0)

        for h in range(N_DEV - 1):
            rdma = pltpu.make_async_remote_copy(
                src_ref=gat_ref.at[h],
                dst_ref=gat_ref.at[h + 1],
                send_sem=send_sems.at[h],
                recv_sem=recv_sems.at[h],
                device_id=(right,),
                device_id_type=pl.DeviceIdType.MESH,
            )
            rdma.start()
            rdma.wait()
            compute_and_store(h + 1, (h + 1) % 2)

    return pl.pallas_call(
        body,
        out_shape=jax.ShapeDtypeStruct((M, n), jnp.float32),
        in_specs=[
            pl.BlockSpec(memory_space=pltpu.VMEM),
            pl.BlockSpec(memory_space=pltpu.VMEM),
        ],
        out_specs=pl.BlockSpec(memory_space=pltpu.ANY),
        scratch_shapes=[
            pltpu.VMEM((N_DEV, m_per, k), jnp.float32),
            pltpu.VMEM((2, m_per, n), jnp.float32),
            pltpu.SemaphoreType.DMA((N_DEV - 1,)),
            pltpu.SemaphoreType.DMA((N_DEV - 1,)),
            pltpu.SemaphoreType.DMA((2,)),
        ],
        compiler_params=pltpu.CompilerParams(collective_id=0),
    )(A, B)


# baseline (device time: 586385 ns/iter reference)
import jax
import jax.numpy as jnp
from jax import lax
from jax.experimental import pallas as pl
from jax.experimental.pallas import tpu as pltpu

N_DEV = 32


def kernel(A, B):
    m_per, k = A.shape
    _, n = B.shape
    M = N_DEV * m_per

    def body(a_ref, b_ref, out_ref, gat_ref, cbuf_ref, send_sems, recv_sems, out_sems):
        my_pos = lax.axis_index("i")
        left = lax.rem(my_pos + N_DEV - 1, N_DEV)
        right = lax.rem(my_pos + 1, N_DEV)

        barrier_sem = pltpu.get_barrier_semaphore()
        for nbr in (left, right):
            pl.semaphore_signal(
                barrier_sem, inc=1,
                device_id=(nbr,), device_id_type=pl.DeviceIdType.MESH,
            )
        pl.semaphore_wait(barrier_sem, 2)

        gat_ref[0, :, :] = a_ref[:, :]

        def compute_and_store(slot, cslot):
            origin = lax.rem(my_pos + N_DEV - slot, N_DEV)
            cbuf_ref[cslot, :, :] = jnp.dot(
                gat_ref[slot, :, :], b_ref[:, :],
                preferred_element_type=jnp.float32,
            )
            cp = pltpu.make_async_copy(
                cbuf_ref.at[cslot],
                out_ref.at[pl.ds(origin * m_per, m_per), :],
                out_sems.at[cslot],
            )
            cp.start()
            cp.wait()

        compute_and_store(0, 0)

        for h in range(N_DEV - 1):
            rdma = pltpu.make_async_remote_copy(
                src_ref=gat_ref.at[h],
                dst_ref=gat_ref.at[h + 1],
                send_sem=send_sems.at[h],
                recv_sem=recv_sems.at[h],
                device_id=(right,),
                device_id_type=pl.DeviceIdType.MESH,
            )
            rdma.start()
            rdma.wait()
            compute_and_store(h + 1, (h + 1) % 2)

    return pl.pallas_call(
        body,
        out_shape=jax.ShapeDtypeStruct((M, n), jnp.float32),
        in_specs=[
            pl.BlockSpec(memory_space=pltpu.VMEM),
            pl.BlockSpec(memory_space=pltpu.VMEM),
        ],
        out_specs=pl.BlockSpec(memory_space=pl.ANY),
        scratch_shapes=[
            pltpu.VMEM((N_DEV, m_per, k), jnp.float32),
            pltpu.VMEM((2, m_per, n), jnp.float32),
            pltpu.SemaphoreType.DMA((N_DEV - 1,)),
            pltpu.SemaphoreType.DMA((N_DEV - 1,)),
            pltpu.SemaphoreType.DMA((2,)),
        ],
        compiler_params=pltpu.CompilerParams(
            collective_id=0,
            vmem_limit_bytes=56 * 1024 * 1024,
        ),
    )(A, B)


# device time: 507600 ns/iter; 1.1552x vs baseline; 1.1552x over previous
import jax
import jax.numpy as jnp
from jax import lax
from jax.experimental import pallas as pl
from jax.experimental.pallas import tpu as pltpu

N_DEV = 32


def kernel(A, B):
    m_per, k = A.shape
    _, n = B.shape
    M = N_DEV * m_per

    def body(a_ref, b_ref, out_ref, gat_ref, cbuf_ref, send_sems, recv_sems, out_sems):
        my_pos = lax.axis_index("i")
        left = lax.rem(my_pos + N_DEV - 1, N_DEV)
        right = lax.rem(my_pos + 1, N_DEV)

        barrier_sem = pltpu.get_barrier_semaphore()
        for nbr in (left, right):
            pl.semaphore_signal(
                barrier_sem, inc=1,
                device_id=(nbr,), device_id_type=pl.DeviceIdType.MESH,
            )
        pl.semaphore_wait(barrier_sem, 2)

        gat_ref[0, :, :] = a_ref[:, :]

        def compute_and_store(slot, cslot):
            origin = lax.rem(my_pos + N_DEV - slot, N_DEV)
            cbuf_ref[cslot, :, :] = jnp.dot(
                gat_ref[slot, :, :], b_ref[:, :],
                preferred_element_type=jnp.float32,
            )
            cp = pltpu.make_async_copy(
                cbuf_ref.at[cslot],
                out_ref.at[pl.ds(origin * m_per, m_per), :],
                out_sems.at[cslot],
            )
            cp.start()
            cp.wait()

        for h in range(N_DEV - 1):
            rdma = pltpu.make_async_remote_copy(
                src_ref=gat_ref.at[h],
                dst_ref=gat_ref.at[h + 1],
                send_sem=send_sems.at[h],
                recv_sem=recv_sems.at[h],
                device_id=(right,),
                device_id_type=pl.DeviceIdType.MESH,
            )
            rdma.start()
            compute_and_store(h, h % 2)
            rdma.wait()
        compute_and_store(N_DEV - 1, (N_DEV - 1) % 2)

    return pl.pallas_call(
        body,
        out_shape=jax.ShapeDtypeStruct((M, n), jnp.float32),
        in_specs=[
            pl.BlockSpec(memory_space=pltpu.VMEM),
            pl.BlockSpec(memory_space=pltpu.VMEM),
        ],
        out_specs=pl.BlockSpec(memory_space=pl.ANY),
        scratch_shapes=[
            pltpu.VMEM((N_DEV, m_per, k), jnp.float32),
            pltpu.VMEM((2, m_per, n), jnp.float32),
            pltpu.SemaphoreType.DMA((N_DEV - 1,)),
            pltpu.SemaphoreType.DMA((N_DEV - 1,)),
            pltpu.SemaphoreType.DMA((2,)),
        ],
        compiler_params=pltpu.CompilerParams(
            collective_id=0,
            vmem_limit_bytes=56 * 1024 * 1024,
        ),
    )(A, B)


# device time: 485327 ns/iter; 1.2082x vs baseline; 1.0459x over previous
import jax
import jax.numpy as jnp
from jax import lax
from jax.experimental import pallas as pl
from jax.experimental.pallas import tpu as pltpu

N_DEV = 32
R_HOPS = N_DEV // 2
L_HOPS = (N_DEV - 1) // 2


def kernel(A, B):
    m_per, k = A.shape
    _, n = B.shape
    M = N_DEV * m_per

    def body(a_ref, b_ref, out_ref, gatR_ref, gatL_ref, cbuf_ref,
             sendR_sems, recvR_sems, sendL_sems, recvL_sems, out_sems):
        my_pos = lax.axis_index("i")
        left = lax.rem(my_pos + N_DEV - 1, N_DEV)
        right = lax.rem(my_pos + 1, N_DEV)

        barrier_sem = pltpu.get_barrier_semaphore()
        for nbr in (left, right):
            pl.semaphore_signal(
                barrier_sem, inc=1,
                device_id=(nbr,), device_id_type=pl.DeviceIdType.MESH,
            )
        pl.semaphore_wait(barrier_sem, 2)

        gatR_ref[0, :, :] = a_ref[:, :]
        gatL_ref[0, :, :] = a_ref[:, :]

        def compute_and_store(gat_ref, slot, sign, cslot):
            origin = lax.rem(my_pos + N_DEV + sign * slot, N_DEV)
            cbuf_ref[cslot, :, :] = jnp.dot(
                gat_ref[slot, :, :], b_ref[:, :],
                preferred_element_type=jnp.float32,
            )
            cp = pltpu.make_async_copy(
                cbuf_ref.at[cslot],
                out_ref.at[pl.ds(origin * m_per, m_per), :],
                out_sems.at[cslot],
            )
            cp.start()
            cp.wait()

        def hop(gat_ref, h, send_sems, recv_sems, nbr):
            return pltpu.make_async_remote_copy(
                src_ref=gat_ref.at[h],
                dst_ref=gat_ref.at[h + 1],
                send_sem=send_sems.at[h],
                recv_sem=recv_sems.at[h],
                device_id=(nbr,),
                device_id_type=pl.DeviceIdType.MESH,
            )

        for h in range(R_HOPS):
            rdmaR = hop(gatR_ref, h, sendR_sems, recvR_sems, right)
            rdmaR.start()
            rdmaL = None
            if h < L_HOPS:
                rdmaL = hop(gatL_ref, h, sendL_sems, recvL_sems, left)
                rdmaL.start()
            compute_and_store(gatR_ref, h, -1, 0)
            if 1 <= h <= L_HOPS:
                compute_and_store(gatL_ref, h, +1, 1)
            rdmaR.wait()
            if rdmaL is not None:
                rdmaL.wait()

        compute_and_store(gatR_ref, R_HOPS, -1, 0)
        compute_and_store(gatL_ref, L_HOPS, +1, 1)

    return pl.pallas_call(
        body,
        out_shape=jax.ShapeDtypeStruct((M, n), jnp.float32),
        in_specs=[
            pl.BlockSpec(memory_space=pltpu.VMEM),
            pl.BlockSpec(memory_space=pltpu.VMEM),
        ],
        out_specs=pl.BlockSpec(memory_space=pl.ANY),
        scratch_shapes=[
            pltpu.VMEM((R_HOPS + 1, m_per, k), jnp.float32),
            pltpu.VMEM((L_HOPS + 1, m_per, k), jnp.float32),
            pltpu.VMEM((2, m_per, n), jnp.float32),
            pltpu.SemaphoreType.DMA((R_HOPS,)),
            pltpu.SemaphoreType.DMA((R_HOPS,)),
            pltpu.SemaphoreType.DMA((L_HOPS,)),
            pltpu.SemaphoreType.DMA((L_HOPS,)),
            pltpu.SemaphoreType.DMA((2,)),
        ],
        compiler_params=pltpu.CompilerParams(
            collective_id=0,
            vmem_limit_bytes=56 * 1024 * 1024,
        ),
    )(A, B)


# device time: 292025 ns/iter; 2.0080x vs baseline; 1.6619x over previous
import jax
import jax.numpy as jnp
from jax import lax
from jax.experimental import pallas as pl
from jax.experimental.pallas import tpu as pltpu

N_DEV = 32
R_HOPS = N_DEV // 2
L_HOPS = (N_DEV - 1) // 2


def _logical_coords():
    order = []
    for z in range(4):
        for y in range(4):
            for x in ([0, 1] if y % 2 == 0 else [1, 0]):
                order.append((x, y, z))
    return order


def _hamiltonian_cycle():
    p_yz = [(0, 0), (1, 0), (2, 0), (3, 0), (3, 1), (3, 2), (3, 3),
            (2, 3), (2, 2), (2, 1), (1, 1), (1, 2), (1, 3), (0, 3),
            (0, 2), (0, 1)]
    cyc = [(0, y, z) for (y, z) in p_yz] + [(1, y, z) for (y, z) in reversed(p_yz)]
    for a, b in zip(cyc, cyc[1:] + cyc[:1]):
        assert sum(abs(u - v) for u, v in zip(a, b)) == 1, (a, b)
    return cyc


_COORDS = _logical_coords()
_COORD_TO_LOG = {c: i for i, c in enumerate(_COORDS)}
PERM = [_COORD_TO_LOG[c] for c in _hamiltonian_cycle()]
INV = [0] * N_DEV
for _j, _l in enumerate(PERM):
    INV[_l] = _j
NEXT = [PERM[(INV[l] + 1) % N_DEV] for l in range(N_DEV)]
PREV = [PERM[(INV[l] - 1) % N_DEV] for l in range(N_DEV)]


def _lookup(table, idx):
    out = jnp.int32(table[0])
    for k in range(1, len(table)):
        out = jnp.where(idx == k, jnp.int32(table[k]), out)
    return out


def kernel(A, B):
    m_per, k = A.shape
    _, n = B.shape
    M = N_DEV * m_per

    def body(a_ref, b_ref, out_ref, gatR_ref, gatL_ref, cbuf_ref,
             sendR_sems, recvR_sems, sendL_sems, recvL_sems, out_sems):
        my_pos = lax.axis_index("i")
        r = _lookup(INV, my_pos)
        right = _lookup(NEXT, my_pos)
        left = _lookup(PREV, my_pos)

        barrier_sem = pltpu.get_barrier_semaphore()
        for nbr in (left, right):
            pl.semaphore_signal(
                barrier_sem, inc=1,
                device_id=(nbr,), device_id_type=pl.DeviceIdType.MESH,
            )
        pl.semaphore_wait(barrier_sem, 2)

        gatR_ref[0, :, :] = a_ref[:, :]
        gatL_ref[0, :, :] = a_ref[:, :]

        def compute_and_store(gat_ref, slot, sign, cslot):
            o_ring = lax.rem(r + N_DEV + sign * slot, N_DEV)
            origin = _lookup(PERM, o_ring)
            cbuf_ref[cslot, :, :] = jnp.dot(
                gat_ref[slot, :, :], b_ref[:, :],
                preferred_element_type=jnp.float32,
            )
            cp = pltpu.make_async_copy(
                cbuf_ref.at[cslot],
                out_ref.at[pl.ds(origin * m_per, m_per), :],
                out_sems.at[cslot],
            )
            cp.start()
            cp.wait()

        def hop(gat_ref, h, send_sems, recv_sems, nbr):
            return pltpu.make_async_remote_copy(
                src_ref=gat_ref.at[h],
                dst_ref=gat_ref.at[h + 1],
                send_sem=send_sems.at[h],
                recv_sem=recv_sems.at[h],
                device_id=(nbr,),
                device_id_type=pl.DeviceIdType.MESH,
            )

        for h in range(R_HOPS):
            rdmaR = hop(gatR_ref, h, sendR_sems, recvR_sems, right)
            rdmaR.start()
            rdmaL = None
            if h < L_HOPS:
                rdmaL = hop(gatL_ref, h, sendL_sems, recvL_sems, left)
                rdmaL.start()
            compute_and_store(gatR_ref, h, -1, 0)
            if 1 <= h <= L_HOPS:
                compute_and_store(gatL_ref, h, +1, 1)
            rdmaR.wait()
            if rdmaL is not None:
                rdmaL.wait()

        compute_and_store(gatR_ref, R_HOPS, -1, 0)
        compute_and_store(gatL_ref, L_HOPS, +1, 1)

    return pl.pallas_call(
        body,
        out_shape=jax.ShapeDtypeStruct((M, n), jnp.float32),
        in_specs=[
            pl.BlockSpec(memory_space=pltpu.VMEM),
            pl.BlockSpec(memory_space=pltpu.VMEM),
        ],
        out_specs=pl.BlockSpec(memory_space=pl.ANY),
        scratch_shapes=[
            pltpu.VMEM((R_HOPS + 1, m_per, k), jnp.float32),
            pltpu.VMEM((L_HOPS + 1, m_per, k), jnp.float32),
            pltpu.VMEM((2, m_per, n), jnp.float32),
            pltpu.SemaphoreType.DMA((R_HOPS,)),
            pltpu.SemaphoreType.DMA((R_HOPS,)),
            pltpu.SemaphoreType.DMA((L_HOPS,)),
            pltpu.SemaphoreType.DMA((L_HOPS,)),
            pltpu.SemaphoreType.DMA((2,)),
        ],
        compiler_params=pltpu.CompilerParams(
            collective_id=0,
            vmem_limit_bytes=56 * 1024 * 1024,
        ),
    )(A, B)


# device time: 288516 ns/iter; 2.0324x vs baseline; 1.0122x over previous
import jax
import jax.numpy as jnp
from jax import lax
from jax.experimental import pallas as pl
from jax.experimental.pallas import tpu as pltpu

N_DEV = 32
R_HOPS = N_DEV // 2
L_HOPS = (N_DEV - 1) // 2


def _logical_coords():
    order = []
    for z in range(4):
        for y in range(4):
            for x in ([0, 1] if y % 2 == 0 else [1, 0]):
                order.append((x, y, z))
    return order


def _hamiltonian_cycle():
    p_yz = [(0, 0), (1, 0), (2, 0), (3, 0), (3, 1), (3, 2), (3, 3),
            (2, 3), (2, 2), (2, 1), (1, 1), (1, 2), (1, 3), (0, 3),
            (0, 2), (0, 1)]
    cyc = [(0, y, z) for (y, z) in p_yz] + [(1, y, z) for (y, z) in reversed(p_yz)]
    for a, b in zip(cyc, cyc[1:] + cyc[:1]):
        assert sum(abs(u - v) for u, v in zip(a, b)) == 1, (a, b)
    return cyc


_COORDS = _logical_coords()
_COORD_TO_LOG = {c: i for i, c in enumerate(_COORDS)}
PERM = [_COORD_TO_LOG[c] for c in _hamiltonian_cycle()]
INV = [0] * N_DEV
for _j, _l in enumerate(PERM):
    INV[_l] = _j
NEXT = [PERM[(INV[l] + 1) % N_DEV] for l in range(N_DEV)]
PREV = [PERM[(INV[l] - 1) % N_DEV] for l in range(N_DEV)]


def _lookup(table, idx):
    out = jnp.int32(table[0])
    for k in range(1, len(table)):
        out = jnp.where(idx == k, jnp.int32(table[k]), out)
    return out


def kernel(A, B):
    m_per, k = A.shape
    _, n = B.shape
    M = N_DEV * m_per

    def body(a_ref, b_ref, out_ref, gatR_ref, gatL_ref, cbuf_ref,
             sendR_sems, recvR_sems, sendL_sems, recvL_sems, out_sems):
        my_pos = lax.axis_index("i")
        r = _lookup(INV, my_pos)
        right = _lookup(NEXT, my_pos)
        left = _lookup(PREV, my_pos)

        barrier_sem = pltpu.get_barrier_semaphore()
        for nbr in (left, right):
            pl.semaphore_signal(
                barrier_sem, inc=1,
                device_id=(nbr,), device_id_type=pl.DeviceIdType.MESH,
            )
        pl.semaphore_wait(barrier_sem, 2)

        gatR_ref[0, :, :] = a_ref[:, :]
        gatL_ref[0, :, :] = a_ref[:, :]

        pending = [None] * 4

        def compute_and_store(gat_ref, slot, sign, cslot):
            o_ring = lax.rem(r + N_DEV + sign * slot, N_DEV)
            origin = _lookup(PERM, o_ring)
            if pending[cslot] is not None:
                pending[cslot].wait()
            cbuf_ref[cslot, :, :] = jnp.dot(
                gat_ref[slot, :, :], b_ref[:, :],
                preferred_element_type=jnp.float32,
            )
            cp = pltpu.make_async_copy(
                cbuf_ref.at[cslot],
                out_ref.at[pl.ds(origin * m_per, m_per), :],
                out_sems.at[cslot],
            )
            cp.start()
            pending[cslot] = cp

        def hop(gat_ref, h, send_sems, recv_sems, nbr):
            return pltpu.make_async_remote_copy(
                src_ref=gat_ref.at[h],
                dst_ref=gat_ref.at[h + 1],
                send_sem=send_sems.at[h],
                recv_sem=recv_sems.at[h],
                device_id=(nbr,),
                device_id_type=pl.DeviceIdType.MESH,
            )

        rdmasR, rdmasL = [], []
        for h in range(R_HOPS):
            rdmaR = hop(gatR_ref, h, sendR_sems, recvR_sems, right)
            rdmaR.start()
            rdmasR.append(rdmaR)
            if h < L_HOPS:
                rdmaL = hop(gatL_ref, h, sendL_sems, recvL_sems, left)
                rdmaL.start()
                rdmasL.append(rdmaL)
            compute_and_store(gatR_ref, h, -1, 2 * (h % 2))
            if 1 <= h <= L_HOPS:
                compute_and_store(gatL_ref, h, +1, 2 * (h % 2) + 1)
            rdmaR.wait_recv()
            if h < L_HOPS:
                rdmaL.wait_recv()

        compute_and_store(gatR_ref, R_HOPS, -1, 2 * (R_HOPS % 2))

        for rdma in rdmasR + rdmasL:
            rdma.wait_send()
        for cp in pending:
            if cp is not None:
                cp.wait()

    return pl.pallas_call(
        body,
        out_shape=jax.ShapeDtypeStruct((M, n), jnp.float32),
        in_specs=[
            pl.BlockSpec(memory_space=pltpu.VMEM),
            pl.BlockSpec(memory_space=pltpu.VMEM),
        ],
        out_specs=pl.BlockSpec(memory_space=pl.ANY),
        scratch_shapes=[
            pltpu.VMEM((R_HOPS + 1, m_per, k), jnp.float32),
            pltpu.VMEM((L_HOPS + 1, m_per, k), jnp.float32),
            pltpu.VMEM((4, m_per, n), jnp.float32),
            pltpu.SemaphoreType.DMA((R_HOPS,)),
            pltpu.SemaphoreType.DMA((R_HOPS,)),
            pltpu.SemaphoreType.DMA((L_HOPS,)),
            pltpu.SemaphoreType.DMA((L_HOPS,)),
            pltpu.SemaphoreType.DMA((4,)),
        ],
        compiler_params=pltpu.CompilerParams(
            collective_id=0,
            vmem_limit_bytes=56 * 1024 * 1024,
        ),
    )(A, B)


# device time: 288206 ns/iter; 2.0346x vs baseline; 1.0011x over previous
import jax
import jax.numpy as jnp
from jax import lax
from jax.experimental import pallas as pl
from jax.experimental.pallas import tpu as pltpu

N_DEV = 32
R_HOPS = N_DEV // 2
L_HOPS = (N_DEV - 1) // 2


def _logical_coords():
    order = []
    for z in range(4):
        for y in range(4):
            for x in ([0, 1] if y % 2 == 0 else [1, 0]):
                order.append((x, y, z))
    return order


def _hamiltonian_cycle():
    p_yz = [(0, 0), (1, 0), (2, 0), (3, 0), (3, 1), (3, 2), (3, 3),
            (2, 3), (2, 2), (2, 1), (1, 1), (1, 2), (1, 3), (0, 3),
            (0, 2), (0, 1)]
    cyc = [(0, y, z) for (y, z) in p_yz] + [(1, y, z) for (y, z) in reversed(p_yz)]
    for a, b in zip(cyc, cyc[1:] + cyc[:1]):
        assert sum(abs(u - v) for u, v in zip(a, b)) == 1, (a, b)
    return cyc


_COORDS = _logical_coords()
_COORD_TO_LOG = {c: i for i, c in enumerate(_COORDS)}
PERM = [_COORD_TO_LOG[c] for c in _hamiltonian_cycle()]
INV = [0] * N_DEV
for _j, _l in enumerate(PERM):
    INV[_l] = _j
NEXT = [PERM[(INV[l] + 1) % N_DEV] for l in range(N_DEV)]
PREV = [PERM[(INV[l] - 1) % N_DEV] for l in range(N_DEV)]


def _lookup(table, idx):
    out = jnp.int32(table[0])
    for k in range(1, len(table)):
        out = jnp.where(idx == k, jnp.int32(table[k]), out)
    return out


def kernel(A, B):
    m_per, k = A.shape
    _, n = B.shape
    M = N_DEV * m_per

    def body(a_ref, b_ref, out_ref, gatR_ref, gatL_ref, cbuf_ref,
             sendR_sems, recvR_sems, sendL_sems, recvL_sems, out_sems):
        my_pos = lax.axis_index("i")
        r = _lookup(INV, my_pos)
        right = _lookup(NEXT, my_pos)
        left = _lookup(PREV, my_pos)

        barrier_sem = pltpu.get_barrier_semaphore()
        for nbr in (left, right):
            pl.semaphore_signal(
                barrier_sem, inc=1,
                device_id=(nbr,), device_id_type=pl.DeviceIdType.MESH,
            )
        pl.semaphore_wait(barrier_sem, 2)

        gatR_ref[0, :, :] = a_ref[:, :]
        gatL_ref[0, :, :] = a_ref[:, :]

        pending = [None] * 4

        def compute_and_store(gat_ref, slot, sign, cslot):
            o_ring = lax.rem(r + N_DEV + sign * slot, N_DEV)
            origin = _lookup(PERM, o_ring)
            if pending[cslot] is not None:
                pending[cslot].wait()
            cbuf_ref[cslot, :, :] = jnp.dot(
                gat_ref[slot, :, :], b_ref[:, :],
                preferred_element_type=jnp.float32,
            )
            cp = pltpu.make_async_copy(
                cbuf_ref.at[cslot],
                out_ref.at[pl.ds(origin * m_per, m_per), :],
                out_sems.at[cslot],
            )
            cp.start()
            pending[cslot] = cp

        def hop(gat_ref, h, send_sems, recv_sems, nbr):
            return pltpu.make_async_remote_copy(
                src_ref=gat_ref.at[h],
                dst_ref=gat_ref.at[h + 1],
                send_sem=send_sems.at[h],
                recv_sem=recv_sems.at[h],
                device_id=(nbr,),
                device_id_type=pl.DeviceIdType.MESH,
            )

        rdmasR, rdmasL = [], []
        for h in range(R_HOPS):
            rdmaR = hop(gatR_ref, h, sendR_sems, recvR_sems, right)
            rdmaR.start()
            rdmasR.append(rdmaR)
            if h < L_HOPS:
                rdmaL = hop(gatL_ref, h, sendL_sems, recvL_sems, left)
                rdmaL.start()
                rdmasL.append(rdmaL)
            if h == 0:
                compute_and_store(gatR_ref, h, -1, 2 * (h % 2))
            rdmaR.wait_recv()
            if h < L_HOPS:
                rdmaL.wait_recv()

        compute_and_store(gatR_ref, R_HOPS, -1, 2 * (R_HOPS % 2))

        for rdma in rdmasR + rdmasL:
            rdma.wait_send()
        for cp in pending:
            if cp is not None:
                cp.wait()

    return pl.pallas_call(
        body,
        out_shape=jax.ShapeDtypeStruct((M, n), jnp.float32),
        in_specs=[
            pl.BlockSpec(memory_space=pltpu.VMEM),
            pl.BlockSpec(memory_space=pltpu.VMEM),
        ],
        out_specs=pl.BlockSpec(memory_space=pl.ANY),
        scratch_shapes=[
            pltpu.VMEM((R_HOPS + 1, m_per, k), jnp.float32),
            pltpu.VMEM((L_HOPS + 1, m_per, k), jnp.float32),
            pltpu.VMEM((4, m_per, n), jnp.float32),
            pltpu.SemaphoreType.DMA((R_HOPS,)),
            pltpu.SemaphoreType.DMA((R_HOPS,)),
            pltpu.SemaphoreType.DMA((L_HOPS,)),
            pltpu.SemaphoreType.DMA((L_HOPS,)),
            pltpu.SemaphoreType.DMA((4,)),
        ],
        compiler_params=pltpu.CompilerParams(
            collective_id=0,
            vmem_limit_bytes=56 * 1024 * 1024,
        ),
    )(A, B)


# device time: 260736 ns/iter; 2.2490x vs baseline; 1.1054x over previous
import jax
import jax.numpy as jnp
from jax import lax
from jax.experimental import pallas as pl
from jax.experimental.pallas import tpu as pltpu

N_DEV = 32
R_HOPS = N_DEV // 2
L_HOPS = (N_DEV - 1) // 2


def _logical_coords():
    order = []
    for z in range(4):
        for y in range(4):
            for x in ([0, 1] if y % 2 == 0 else [1, 0]):
                order.append((x, y, z))
    return order


def _hamiltonian_cycle():
    p_yz = [(0, 0), (1, 0), (2, 0), (3, 0), (3, 1), (3, 2), (3, 3),
            (2, 3), (2, 2), (2, 1), (1, 1), (1, 2), (1, 3), (0, 3),
            (0, 2), (0, 1)]
    cyc = [(0, y, z) for (y, z) in p_yz] + [(1, y, z) for (y, z) in reversed(p_yz)]
    for a, b in zip(cyc, cyc[1:] + cyc[:1]):
        assert sum(abs(u - v) for u, v in zip(a, b)) == 1, (a, b)
    return cyc


_COORDS = _logical_coords()
_COORD_TO_LOG = {c: i for i, c in enumerate(_COORDS)}
PERM = [_COORD_TO_LOG[c] for c in _hamiltonian_cycle()]
INV = [0] * N_DEV
for _j, _l in enumerate(PERM):
    INV[_l] = _j
NEXT = [PERM[(INV[l] + 1) % N_DEV] for l in range(N_DEV)]
PREV = [PERM[(INV[l] - 1) % N_DEV] for l in range(N_DEV)]


def _lookup(table, idx):
    out = jnp.int32(table[0])
    for k in range(1, len(table)):
        out = jnp.where(idx == k, jnp.int32(table[k]), out)
    return out


def kernel(A, B):
    m_per, k = A.shape
    _, n = B.shape
    M = N_DEV * m_per

    def body(a_ref, b_ref, out_ref, gatR_ref, gatL_ref, cbuf_ref,
             sendR_sems, recvR_sems, sendL_sems, recvL_sems, out_sems):
        my_pos = lax.axis_index("i")
        r = _lookup(INV, my_pos)
        right = _lookup(NEXT, my_pos)
        left = _lookup(PREV, my_pos)

        barrier_sem = pltpu.get_barrier_semaphore()
        for nbr in (left, right):
            pl.semaphore_signal(
                barrier_sem, inc=1,
                device_id=(nbr,), device_id_type=pl.DeviceIdType.MESH,
            )
        pl.semaphore_wait(barrier_sem, 2)

        gatR_ref[0, :, :] = a_ref[:, :]
        gatL_ref[0, :, :] = a_ref[:, :]

        pending = [None] * 4

        def compute_and_store(gat_ref, slot, sign, cslot):
            o_ring = lax.rem(r + N_DEV + sign * slot, N_DEV)
            origin = _lookup(PERM, o_ring)
            if pending[cslot] is not None:
                pending[cslot].wait()
            cbuf_ref[cslot, :, :] = jnp.dot(
                gat_ref[slot, :, :], b_ref[:, :],
                preferred_element_type=jnp.float32,
            )
            cp = pltpu.make_async_copy(
                cbuf_ref.at[cslot],
                out_ref.at[pl.ds(origin * m_per, m_per), :],
                out_sems.at[cslot],
            )
            cp.start()
            pending[cslot] = cp

        halfm = m_per // 2

        def hop_half(gat_ref, s, send_sems, recv_sems, nbr):
            h, q = divmod(s, 2)
            return pltpu.make_async_remote_copy(
                src_ref=gat_ref.at[h, pl.ds(q * halfm, halfm), :],
                dst_ref=gat_ref.at[h + 1, pl.ds(q * halfm, halfm), :],
                send_sem=send_sems.at[s],
                recv_sem=recv_sems.at[s],
                device_id=(nbr,),
                device_id_type=pl.DeviceIdType.MESH,
            )

        nR, nL = 2 * R_HOPS, 2 * L_HOPS
        rdmasR, rdmasL = [], []
        for s in (0, 1):
            rdmasR.append(hop_half(gatR_ref, s, sendR_sems, recvR_sems, right))
            rdmasR[-1].start()
            rdmasL.append(hop_half(gatL_ref, s, sendL_sems, recvL_sems, left))
            rdmasL[-1].start()
        compute_and_store(gatR_ref, 0, -1, 0)

        for s in range(2, nR):
            t = s - 2
            rdmasR[t].wait_recv()
            rdmasR.append(hop_half(gatR_ref, s, sendR_sems, recvR_sems, right))
            rdmasR[-1].start()
            if t < nL:
                rdmasL[t].wait_recv()
            if s < nL:
                rdmasL.append(hop_half(gatL_ref, s, sendL_sems, recvL_sems, left))
                rdmasL[-1].start()
            if t % 2 == 1:
                c = (t + 1) // 2
                compute_and_store(gatR_ref, c, -1, 2 * (c % 2))
                if t < nL:
                    compute_and_store(gatL_ref, c, +1, 2 * (c % 2) + 1)

        rdmasR[nR - 2].wait_recv()
        rdmasR[nR - 1].wait_recv()
        compute_and_store(gatR_ref, R_HOPS, -1, 2 * (R_HOPS % 2))

        for rdma in rdmasR + rdmasL:
            rdma.wait_send()
        for cp in pending:
            if cp is not None:
                cp.wait()

    return pl.pallas_call(
        body,
        out_shape=jax.ShapeDtypeStruct((M, n), jnp.float32),
        in_specs=[
            pl.BlockSpec(memory_space=pltpu.VMEM),
            pl.BlockSpec(memory_space=pltpu.VMEM),
        ],
        out_specs=pl.BlockSpec(memory_space=pl.ANY),
        scratch_shapes=[
            pltpu.VMEM((R_HOPS + 1, m_per, k), jnp.float32),
            pltpu.VMEM((L_HOPS + 1, m_per, k), jnp.float32),
            pltpu.VMEM((4, m_per, n), jnp.float32),
            pltpu.SemaphoreType.DMA((2 * R_HOPS,)),
            pltpu.SemaphoreType.DMA((2 * R_HOPS,)),
            pltpu.SemaphoreType.DMA((2 * L_HOPS,)),
            pltpu.SemaphoreType.DMA((2 * L_HOPS,)),
            pltpu.SemaphoreType.DMA((4,)),
        ],
        compiler_params=pltpu.CompilerParams(
            collective_id=0,
            vmem_limit_bytes=56 * 1024 * 1024,
        ),
    )(A, B)
